# baseline (speedup 1.0000x reference)
import functools

import jax
import jax.numpy as jnp
from jax import lax
from jax.experimental import pallas as pl
from jax.experimental.pallas import tpu as pltpu
from jax.experimental.pallas import tpu_sc as plsc

E = 8
NG = 2
BLK = 128
NW = 32
LANES = 16


def _pack_bf16(lo_f32, hi_f32):
    lo = lax.bitcast_convert_type(
        lo_f32.astype(jnp.bfloat16).astype(jnp.float32), jnp.int32)
    hi = lax.bitcast_convert_type(
        hi_f32.astype(jnp.bfloat16).astype(jnp.float32), jnp.int32)
    return lax.shift_right_logical(lo, 16) | (hi & jnp.int32(-65536))


def _unpack_bf16(p):
    lo = lax.bitcast_convert_type(lax.shift_left(p, 16), jnp.float32)
    hi = lax.bitcast_convert_type(p & jnp.int32(-65536), jnp.float32)
    return lo, hi


def _gate_route_body(T, NB, x_ref, gw_ref, xp_ref, d0_ref, d1_ref, w0_ref,
                     w1_ref, bexp_ref):
    x = x_ref[...]
    H2 = x.shape[1] // 2
    xp_ref[...] = _pack_bf16(x[:, :H2], x[:, H2:])
    lT = lax.dot_general(gw_ref[...], x, (((1,), (1,)), ((), ())),
                         preferred_element_type=jnp.float32)
    eidx = lax.broadcasted_iota(jnp.int32, (E, T), 0)
    gsz = E // NG
    g0 = jnp.max(lT[:gsz], axis=0, keepdims=True)
    g1 = jnp.max(lT[gsz:], axis=0, keepdims=True)
    hi = jnp.where(eidx >= gsz, 1.0, 0.0)
    use1 = jnp.where(g1 > g0, 1.0, 0.0)
    in_grp = use1 * hi + (1.0 - use1) * (1.0 - hi)
    ml = lT - (1.0 - in_grp) * jnp.float32(1e30)
    m1 = jnp.max(ml, axis=0, keepdims=True)
    i1 = jnp.min(jnp.where(ml == m1, eidx, E + 1), axis=0, keepdims=True)
    ml2 = jnp.where(eidx == i1, jnp.float32(-1e30), ml)
    m2 = jnp.max(ml2, axis=0, keepdims=True)
    i2 = jnp.min(jnp.where(ml2 == m2, eidx, E + 1), axis=0, keepdims=True)
    wa = jax.nn.sigmoid(m1 - m2)
    w0_ref[...] = jnp.transpose(jnp.broadcast_to(wa, (LANES, T)), (1, 0))
    w1_ref[...] = jnp.transpose(
        jnp.broadcast_to(1.0 - wa, (LANES, T)), (1, 0))
    oh1 = eidx == i1
    oh2 = eidx == i2
    sel = jnp.where(oh1 | oh2, 1.0, 0.0)
    incl = sel
    off = 1
    while off < T:
        sh = jnp.concatenate(
            [jnp.zeros((E, off), jnp.float32), incl[:, :T - off]], axis=1)
        incl = incl + sh
        off *= 2
    ranks = incl - sel
    counts = incl[:, T - 1:T]
    nblk = jnp.floor((counts + (BLK - 1)) * (1.0 / BLK))
    acc = nblk
    off = 1
    while off < E:
        sh = jnp.concatenate(
            [jnp.zeros((off, 1), jnp.float32), acc[:E - off]], axis=0)
        acc = acc + sh
        off *= 2
    off_blk = acc - nblk
    destf = ranks + off_blk * BLK
    d0_ref[...] = jnp.sum(jnp.where(oh1, destf, 0.0), axis=0).astype(jnp.int32)
    d1_ref[...] = jnp.sum(jnp.where(oh2, destf, 0.0), axis=0).astype(jnp.int32)
    totalf = jnp.sum(nblk, axis=0, keepdims=True)
    bidx = lax.broadcasted_iota(jnp.int32, (E, NB), 1).astype(jnp.float32)
    bidx = jnp.minimum(bidx, totalf - 1.0)
    bexp = jnp.sum(jnp.where(bidx >= off_blk, 1, 0), axis=0) - 1
    total = jnp.sum(nblk, axis=0).astype(jnp.int32)
    bexp_ref[...] = jnp.concatenate([bexp, total], axis=0)


def _gate_route(x, gw, NB):
    T, H = x.shape
    return pl.pallas_call(
        functools.partial(_gate_route_body, T, NB),
        out_shape=(
            jax.ShapeDtypeStruct((T, H // 2), jnp.int32),
            jax.ShapeDtypeStruct((T,), jnp.int32),
            jax.ShapeDtypeStruct((T,), jnp.int32),
            jax.ShapeDtypeStruct((T, LANES), jnp.float32),
            jax.ShapeDtypeStruct((T, LANES), jnp.float32),
            jax.ShapeDtypeStruct((NB + 1,), jnp.int32),
        ),
    )(x, gw)


def _scatter_body(TPW, x_hbm, d0_hbm, d1_hbm, xs_hbm, idx0_v, idx1_v, rows_v,
                  sem):
    wid = lax.axis_index("s") * 2 + lax.axis_index("c")
    base = wid * TPW
    pltpu.sync_copy(d0_hbm.at[pl.ds(base, TPW)], idx0_v)
    pltpu.sync_copy(d1_hbm.at[pl.ds(base, TPW)], idx1_v)
    pltpu.sync_copy(x_hbm.at[pl.ds(base, TPW)], rows_v)
    c0 = pltpu.async_copy(rows_v, xs_hbm.at[idx0_v], sem)
    c1 = pltpu.async_copy(rows_v, xs_hbm.at[idx1_v], sem)
    c0.wait()
    c1.wait()


def _dispatch(xp, d0, d1, NR):
    T, HP = xp.shape
    TPW = T // NW
    mesh = plsc.VectorSubcoreMesh(core_axis_name="c", subcore_axis_name="s")
    return pl.kernel(
        functools.partial(_scatter_body, TPW),
        mesh=mesh,
        out_type=jax.ShapeDtypeStruct((NR, HP), jnp.int32),
        scratch_types=[
            pltpu.VMEM((TPW,), jnp.int32),
            pltpu.VMEM((TPW,), jnp.int32),
            pltpu.VMEM((TPW, HP), jnp.int32),
            pltpu.SemaphoreType.DMA,
        ],
    )(xp, d0, d1)


def _gemm_body(I, NB, be_ref, xs_ref, w13_ref, w2_ref, ys_ref):
    @pl.when(pl.program_id(0) < be_ref[NB])
    def _():
        xlo, xhi = _unpack_bf16(xs_ref[...])
        xb = jnp.concatenate([xlo, xhi], axis=1).astype(jnp.bfloat16)
        g = lax.dot_general(xb, w13_ref[0].astype(jnp.bfloat16),
                            (((1,), (1,)), ((), ())),
                            preferred_element_type=jnp.float32)
        gate = g[:, :I]
        up = g[:, I:]
        h = (gate * jax.nn.sigmoid(gate) * up).astype(jnp.bfloat16)
        ys_ref[...] = lax.dot_general(h, w2_ref[0].astype(jnp.bfloat16),
                                      (((1,), (1,)), ((), ())),
                                      preferred_element_type=jnp.float32)


def _expert_gemm(bexp, xs, w13, w2, NB):
    NR, HP = xs.shape
    I2 = w13.shape[1]
    H = w2.shape[1]
    I = w2.shape[2]
    grid_spec = pltpu.PrefetchScalarGridSpec(
        num_scalar_prefetch=1,
        grid=(NB,),
        in_specs=[
            pl.BlockSpec((BLK, HP), lambda i, be: (jnp.minimum(i, be[NB] - 1), 0)),
            pl.BlockSpec((1, I2, H), lambda i, be: (be[i], 0, 0)),
            pl.BlockSpec((1, H, I), lambda i, be: (be[i], 0, 0)),
        ],
        out_specs=pl.BlockSpec((BLK, H), lambda i, be: (i, 0)),
    )
    return pl.pallas_call(
        functools.partial(_gemm_body, I, NB),
        grid_spec=grid_spec,
        out_shape=jax.ShapeDtypeStruct((NR, H), jnp.float32),
    )(bexp, xs, w13, w2)


def _combine_body(TPW, H, ys_hbm, d0_hbm, d1_hbm, w0_hbm, w1_hbm, out_hbm,
                  idx0_v, idx1_v, r0_v, r1_v, w0_v, w1_v, sem):
    wid = lax.axis_index("s") * 2 + lax.axis_index("c")
    base = wid * TPW
    pltpu.sync_copy(d0_hbm.at[pl.ds(base, TPW)], idx0_v)
    c0 = pltpu.async_copy(ys_hbm.at[idx0_v], r0_v, sem)
    pltpu.sync_copy(d1_hbm.at[pl.ds(base, TPW)], idx1_v)
    c1 = pltpu.async_copy(ys_hbm.at[idx1_v], r1_v, sem)
    pltpu.sync_copy(w0_hbm.at[pl.ds(base, TPW)], w0_v)
    pltpu.sync_copy(w1_hbm.at[pl.ds(base, TPW)], w1_v)
    c0.wait()
    c1.wait()
    nch = H // LANES

    def tok(i, carry):
        wa = w0_v[i, :]
        wb = w1_v[i, :]
        for j in range(nch):
            sl = pl.ds(j * LANES, LANES)
            r0_v[i, sl] = r0_v[i, sl] * wa + r1_v[i, sl] * wb
        return carry

    lax.fori_loop(0, TPW, tok, 0)
    pltpu.sync_copy(r0_v, out_hbm.at[pl.ds(base, TPW)])


def _combine(ys, d0, d1, w0, w1, T):
    NR, H = ys.shape
    TPW = T // NW
    mesh = plsc.VectorSubcoreMesh(core_axis_name="c", subcore_axis_name="s")
    return pl.kernel(
        functools.partial(_combine_body, TPW, H),
        mesh=mesh,
        out_type=jax.ShapeDtypeStruct((T, H), jnp.float32),
        scratch_types=[
            pltpu.VMEM((TPW,), jnp.int32),
            pltpu.VMEM((TPW,), jnp.int32),
            pltpu.VMEM((TPW, H), jnp.float32),
            pltpu.VMEM((TPW, H), jnp.float32),
            pltpu.VMEM((TPW, LANES), jnp.float32),
            pltpu.VMEM((TPW, LANES), jnp.float32),
            pltpu.SemaphoreType.DMA,
        ],
    )(ys, d0, d1, w0, w1)


def kernel(hidden_states, gate_weight, w13_weight, w2_weight):
    b, s, h = hidden_states.shape
    x = hidden_states.reshape(-1, h)
    T = x.shape[0]
    NB = (T * 2) // BLK + (E - 1)
    NR = NB * BLK
    xp, d0, d1, w0, w1, bexp = _gate_route(x, gate_weight, NB)
    xs = _dispatch(xp, d0, d1, NR)
    ys = _expert_gemm(bexp, xs, w13_weight, w2_weight, NB)
    out = _combine(ys, d0, d1, w0, w1, T)
    return out.reshape(b, s, h)


# speedup vs baseline: 1.1465x; 1.1465x over previous
import functools

import jax
import jax.numpy as jnp
from jax import lax
from jax.experimental import pallas as pl
from jax.experimental.pallas import tpu as pltpu
from jax.experimental.pallas import tpu_sc as plsc

E = 8
NG = 2
BLK = 256
NW = 32
LANES = 16


def _pack_bf16(lo_f32, hi_f32):
    lo = lax.bitcast_convert_type(
        lo_f32.astype(jnp.bfloat16).astype(jnp.float32), jnp.int32)
    hi = lax.bitcast_convert_type(
        hi_f32.astype(jnp.bfloat16).astype(jnp.float32), jnp.int32)
    return lax.shift_right_logical(lo, 16) | (hi & jnp.int32(-65536))


def _unpack_bf16(p):
    lo = lax.bitcast_convert_type(lax.shift_left(p, 16), jnp.float32)
    hi = lax.bitcast_convert_type(p & jnp.int32(-65536), jnp.float32)
    return lo, hi


def _gate_route_body(T, NB, x_ref, gw_ref, xp_ref, d0_ref, d1_ref, w0_ref,
                     w1_ref, bexp_ref):
    x = x_ref[...]
    H2 = x.shape[1] // 2
    xp_ref[...] = _pack_bf16(x[:, :H2], x[:, H2:])
    lT = lax.dot_general(gw_ref[...], x, (((1,), (1,)), ((), ())),
                         preferred_element_type=jnp.float32)
    eidx = lax.broadcasted_iota(jnp.int32, (E, T), 0)
    gsz = E // NG
    g0 = jnp.max(lT[:gsz], axis=0, keepdims=True)
    g1 = jnp.max(lT[gsz:], axis=0, keepdims=True)
    hi = jnp.where(eidx >= gsz, 1.0, 0.0)
    use1 = jnp.where(g1 > g0, 1.0, 0.0)
    in_grp = use1 * hi + (1.0 - use1) * (1.0 - hi)
    ml = lT - (1.0 - in_grp) * jnp.float32(1e30)
    m1 = jnp.max(ml, axis=0, keepdims=True)
    i1 = jnp.min(jnp.where(ml == m1, eidx, E + 1), axis=0, keepdims=True)
    ml2 = jnp.where(eidx == i1, jnp.float32(-1e30), ml)
    m2 = jnp.max(ml2, axis=0, keepdims=True)
    i2 = jnp.min(jnp.where(ml2 == m2, eidx, E + 1), axis=0, keepdims=True)
    wa = jax.nn.sigmoid(m1 - m2)
    w0_ref[...] = jnp.transpose(jnp.broadcast_to(wa, (LANES, T)), (1, 0))
    w1_ref[...] = jnp.transpose(
        jnp.broadcast_to(1.0 - wa, (LANES, T)), (1, 0))
    oh1 = eidx == i1
    oh2 = eidx == i2
    sel = jnp.where(oh1 | oh2, 1.0, 0.0)
    incl = sel
    off = 1
    while off < T:
        sh = jnp.concatenate(
            [jnp.zeros((E, off), jnp.float32), incl[:, :T - off]], axis=1)
        incl = incl + sh
        off *= 2
    ranks = incl - sel
    counts = incl[:, T - 1:T]
    nblk = jnp.floor((counts + (BLK - 1)) * (1.0 / BLK))
    acc = nblk
    off = 1
    while off < E:
        sh = jnp.concatenate(
            [jnp.zeros((off, 1), jnp.float32), acc[:E - off]], axis=0)
        acc = acc + sh
        off *= 2
    off_blk = acc - nblk
    destf = ranks + off_blk * BLK
    d0_ref[...] = jnp.sum(jnp.where(oh1, destf, 0.0), axis=0).astype(jnp.int32)
    d1_ref[...] = jnp.sum(jnp.where(oh2, destf, 0.0), axis=0).astype(jnp.int32)
    totalf = jnp.sum(nblk, axis=0, keepdims=True)
    bidx = lax.broadcasted_iota(jnp.int32, (E, NB), 1).astype(jnp.float32)
    bidx = jnp.minimum(bidx, totalf - 1.0)
    bexp = jnp.sum(jnp.where(bidx >= off_blk, 1, 0), axis=0) - 1
    total = jnp.sum(nblk, axis=0).astype(jnp.int32)
    bexp_ref[...] = jnp.concatenate([bexp, total], axis=0)


def _gate_route(x, gw, NB):
    T, H = x.shape
    return pl.pallas_call(
        functools.partial(_gate_route_body, T, NB),
        out_shape=(
            jax.ShapeDtypeStruct((T, H // 2), jnp.int32),
            jax.ShapeDtypeStruct((T,), jnp.int32),
            jax.ShapeDtypeStruct((T,), jnp.int32),
            jax.ShapeDtypeStruct((T, LANES), jnp.float32),
            jax.ShapeDtypeStruct((T, LANES), jnp.float32),
            jax.ShapeDtypeStruct((NB + 1,), jnp.int32),
        ),
    )(x, gw)


def _scatter_body(TPW, x_hbm, d0_hbm, d1_hbm, xs_hbm, idx0_v, idx1_v, rows_v,
                  sem):
    wid = lax.axis_index("s") * 2 + lax.axis_index("c")
    base = wid * TPW
    pltpu.sync_copy(d0_hbm.at[pl.ds(base, TPW)], idx0_v)
    pltpu.sync_copy(d1_hbm.at[pl.ds(base, TPW)], idx1_v)
    pltpu.sync_copy(x_hbm.at[pl.ds(base, TPW)], rows_v)
    c0 = pltpu.async_copy(rows_v, xs_hbm.at[idx0_v], sem)
    c1 = pltpu.async_copy(rows_v, xs_hbm.at[idx1_v], sem)
    c0.wait()
    c1.wait()


def _dispatch(xp, d0, d1, NR):
    T, HP = xp.shape
    TPW = T // NW
    mesh = plsc.VectorSubcoreMesh(core_axis_name="c", subcore_axis_name="s")
    return pl.kernel(
        functools.partial(_scatter_body, TPW),
        mesh=mesh,
        out_type=jax.ShapeDtypeStruct((NR, HP), jnp.int32),
        scratch_types=[
            pltpu.VMEM((TPW,), jnp.int32),
            pltpu.VMEM((TPW,), jnp.int32),
            pltpu.VMEM((TPW, HP), jnp.int32),
            pltpu.SemaphoreType.DMA,
        ],
    )(xp, d0, d1)


def _gemm_body(I, NB, be_ref, xs_ref, w13_ref, w2_ref, ys_ref):
    @pl.when(pl.program_id(0) < be_ref[NB])
    def _():
        xlo, xhi = _unpack_bf16(xs_ref[...])
        xb = jnp.concatenate([xlo, xhi], axis=1).astype(jnp.bfloat16)
        g = lax.dot_general(xb, w13_ref[0].astype(jnp.bfloat16),
                            (((1,), (1,)), ((), ())),
                            preferred_element_type=jnp.float32)
        gate = g[:, :I]
        up = g[:, I:]
        h = (gate * jax.nn.sigmoid(gate) * up).astype(jnp.bfloat16)
        ys_ref[...] = lax.dot_general(h, w2_ref[0].astype(jnp.bfloat16),
                                      (((1,), (1,)), ((), ())),
                                      preferred_element_type=jnp.float32)


def _expert_gemm(bexp, xs, w13, w2, NB):
    NR, HP = xs.shape
    I2 = w13.shape[1]
    H = w2.shape[1]
    I = w2.shape[2]
    grid_spec = pltpu.PrefetchScalarGridSpec(
        num_scalar_prefetch=1,
        grid=(NB,),
        in_specs=[
            pl.BlockSpec((BLK, HP), lambda i, be: (jnp.minimum(i, be[NB] - 1), 0)),
            pl.BlockSpec((1, I2, H), lambda i, be: (be[i], 0, 0)),
            pl.BlockSpec((1, H, I), lambda i, be: (be[i], 0, 0)),
        ],
        out_specs=pl.BlockSpec((BLK, H), lambda i, be: (i, 0)),
    )
    return pl.pallas_call(
        functools.partial(_gemm_body, I, NB),
        grid_spec=grid_spec,
        out_shape=jax.ShapeDtypeStruct((NR, H), jnp.float32),
    )(bexp, xs, w13, w2)


def _combine_body(TPW, H, ys_hbm, d0_hbm, d1_hbm, w0_hbm, w1_hbm, out_hbm,
                  idx0_v, idx1_v, r0_v, r1_v, w0_v, w1_v, sem):
    wid = lax.axis_index("s") * 2 + lax.axis_index("c")
    base = wid * TPW
    pltpu.sync_copy(d0_hbm.at[pl.ds(base, TPW)], idx0_v)
    c0 = pltpu.async_copy(ys_hbm.at[idx0_v], r0_v, sem)
    pltpu.sync_copy(d1_hbm.at[pl.ds(base, TPW)], idx1_v)
    c1 = pltpu.async_copy(ys_hbm.at[idx1_v], r1_v, sem)
    pltpu.sync_copy(w0_hbm.at[pl.ds(base, TPW)], w0_v)
    pltpu.sync_copy(w1_hbm.at[pl.ds(base, TPW)], w1_v)
    c0.wait()
    c1.wait()
    nch = H // LANES

    def tok(i, carry):
        wa = w0_v[i, :]
        wb = w1_v[i, :]
        for j in range(nch):
            sl = pl.ds(j * LANES, LANES)
            r0_v[i, sl] = r0_v[i, sl] * wa + r1_v[i, sl] * wb
        return carry

    lax.fori_loop(0, TPW, tok, 0)
    pltpu.sync_copy(r0_v, out_hbm.at[pl.ds(base, TPW)])


def _combine(ys, d0, d1, w0, w1, T):
    NR, H = ys.shape
    TPW = T // NW
    mesh = plsc.VectorSubcoreMesh(core_axis_name="c", subcore_axis_name="s")
    return pl.kernel(
        functools.partial(_combine_body, TPW, H),
        mesh=mesh,
        out_type=jax.ShapeDtypeStruct((T, H), jnp.float32),
        scratch_types=[
            pltpu.VMEM((TPW,), jnp.int32),
            pltpu.VMEM((TPW,), jnp.int32),
            pltpu.VMEM((TPW, H), jnp.float32),
            pltpu.VMEM((TPW, H), jnp.float32),
            pltpu.VMEM((TPW, LANES), jnp.float32),
            pltpu.VMEM((TPW, LANES), jnp.float32),
            pltpu.SemaphoreType.DMA,
        ],
    )(ys, d0, d1, w0, w1)


def kernel(hidden_states, gate_weight, w13_weight, w2_weight):
    b, s, h = hidden_states.shape
    x = hidden_states.reshape(-1, h)
    T = x.shape[0]
    NB = (T * 2) // BLK + (E - 1)
    NR = NB * BLK
    xp, d0, d1, w0, w1, bexp = _gate_route(x, gate_weight, NB)
    xs = _dispatch(xp, d0, d1, NR)
    ys = _expert_gemm(bexp, xs, w13_weight, w2_weight, NB)
    out = _combine(ys, d0, d1, w0, w1, T)
    return out.reshape(b, s, h)
